# Initial kernel scaffold; baseline (speedup 1.0000x reference)
import jax
import jax.numpy as jnp
from jax.experimental import pallas as pl


def kernel(x, edge_index, W1, b1, W2, b2, W3, b3, lap_weight):
    raise NotImplementedError("write your pallas kernel here")


# speedup vs baseline: 1.6497x; 1.6497x over previous
import functools

import jax
import jax.numpy as jnp
from jax import lax
from jax.experimental import pallas as pl
from jax.experimental.pallas import tpu as pltpu
from jax.experimental.pallas import tpu_sc as plsc

N = 10000
E = 320000
D_IN = 128
D_H = 128
D_OUT = 64

NC = 2
NS = 16
NW = NC * NS
EPT = E // NW
ROWS_PT = N // NS

_MESH = plsc.VectorSubcoreMesh(core_axis_name="c", subcore_axis_name="s")


def _sc_agg(d):
    chunk = 200
    nchunk = EPT // chunk

    @functools.partial(
        pl.kernel,
        out_type=jax.ShapeDtypeStruct((NC * N, d), jnp.float32),
        mesh=_MESH,
        scratch_types=[
            pltpu.VMEM((chunk,), jnp.int32),
            pltpu.VMEM((chunk,), jnp.int32),
            pltpu.VMEM((chunk, d), jnp.float32),
            pltpu.VMEM_SHARED((N, d), jnp.float32),
            pltpu.SemaphoreType.DMA,
        ],
    )
    def k(g_hbm, src_hbm, dst_hbm, out_hbm, srcv, dstv, rows, acc, sem):
        cid = lax.axis_index("c")
        sid = lax.axis_index("s")
        wid = cid * NS + sid

        zero16 = jnp.zeros((16,), jnp.float32)

        def zrow(i, _):
            for j in range(d // 16):
                rows[i, pl.ds(j * 16, 16)] = zero16
            return 0

        lax.fori_loop(0, chunk, zrow, 0)
        w0 = sid * 624
        b0 = 624 * (NS - 1)

        def _zero_acc(base, total):
            off = 0
            while off < total:
                n = min(chunk, total - off)
                pltpu.sync_copy(rows.at[pl.ds(0, n)],
                                acc.at[pl.ds(base + off, n)])
                off += n

        @pl.when(sid < NS - 1)
        def _():
            _zero_acc(w0, 624)

        @pl.when(sid == NS - 1)
        def _():
            _zero_acc(b0, 640)

        plsc.subcore_barrier()

        base = wid * EPT

        def body(ci, _):
            off = base + ci * chunk
            pltpu.sync_copy(src_hbm.at[pl.ds(off, chunk)], srcv)
            pltpu.sync_copy(dst_hbm.at[pl.ds(off, chunk)], dstv)
            pltpu.async_copy(g_hbm.at[srcv], rows, sem).wait()
            pltpu.sync_copy(rows, acc.at[dstv], add=True)
            return 0

        lax.fori_loop(0, nchunk, body, 0)
        plsc.subcore_barrier()

        def _writeout(base, total):
            off = 0
            while off < total:
                n = min(chunk, total - off)
                pltpu.sync_copy(acc.at[pl.ds(base + off, n)],
                                rows.at[pl.ds(0, n)])
                pltpu.sync_copy(rows.at[pl.ds(0, n)],
                                out_hbm.at[pl.ds(cid * N + base + off, n)])
                off += n

        @pl.when(sid < NS - 1)
        def _():
            _writeout(w0, 624)

        @pl.when(sid == NS - 1)
        def _():
            _writeout(b0, 640)

    return k


def _sc_deg():
    chunk = 2000
    nchunk = EPT // chunk

    zlen = 640

    @functools.partial(
        pl.kernel,
        out_type=jax.ShapeDtypeStruct((NC * N,), jnp.float32),
        mesh=_MESH,
        scratch_types=[
            pltpu.VMEM((chunk,), jnp.int32),
            pltpu.VMEM((chunk,), jnp.float32),
            pltpu.VMEM((zlen,), jnp.float32),
            pltpu.VMEM_SHARED((N,), jnp.float32),
        ],
    )
    def k(dst_hbm, out_hbm, dstv, ones, zbuf, acc):
        cid = lax.axis_index("c")
        sid = lax.axis_index("s")
        wid = cid * NS + sid

        one16 = jnp.ones((16,), jnp.float32)
        zero16 = jnp.zeros((16,), jnp.float32)

        def fill(i, _):
            ones[pl.ds(i * 16, 16)] = one16
            return 0

        lax.fori_loop(0, chunk // 16, fill, 0)

        def zfill(i, _):
            zbuf[pl.ds(i * 16, 16)] = zero16
            return 0

        lax.fori_loop(0, zlen // 16, zfill, 0)

        w0 = sid * 624
        wn = N - 624 * (NS - 1)

        @pl.when(sid < NS - 1)
        def _():
            pltpu.sync_copy(zbuf.at[pl.ds(0, 624)], acc.at[pl.ds(w0, 624)])

        @pl.when(sid == NS - 1)
        def _():
            pltpu.sync_copy(zbuf, acc.at[pl.ds(624 * (NS - 1), wn)])

        plsc.subcore_barrier()

        base = wid * EPT

        def body(ci, _):
            off = base + ci * chunk
            pltpu.sync_copy(dst_hbm.at[pl.ds(off, chunk)], dstv)
            pltpu.sync_copy(ones, acc.at[dstv], add=True)
            return 0

        lax.fori_loop(0, nchunk, body, 0)
        plsc.subcore_barrier()

        @pl.when(sid < NS - 1)
        def _():
            pltpu.sync_copy(acc.at[pl.ds(w0, 624)], zbuf.at[pl.ds(0, 624)])
            pltpu.sync_copy(zbuf.at[pl.ds(0, 624)],
                            out_hbm.at[pl.ds(cid * N + w0, 624)])

        @pl.when(sid == NS - 1)
        def _():
            pltpu.sync_copy(acc.at[pl.ds(624 * (NS - 1), wn)], zbuf)
            pltpu.sync_copy(zbuf,
                            out_hbm.at[pl.ds(cid * N + 624 * (NS - 1), wn)])

    return k


def _sc_reg():
    R = 128
    SHIFT = 7
    NSTRIPE = (N + R - 1) // R
    NITER = (NSTRIPE + NC - 1) // NC
    EPT2 = E // NS
    ECH = 2000
    NECH = EPT2 // ECH
    NB = NSTRIPE
    NBP = 80
    KCAP = EPT2 + 16 * NB
    KCAP = ((KCAP + 511) // 512) * 512
    DUMP = R * N
    ACCW = R * N + 64
    ZPT = 80000

    @functools.partial(
        pl.kernel,
        out_type=jax.ShapeDtypeStruct((NW * 16,), jnp.float32),
        mesh=_MESH,
        compiler_params=pltpu.CompilerParams(needs_layout_passes=False),
        scratch_types=[
            pltpu.VMEM((KCAP,), jnp.int32),
            pltpu.VMEM((KCAP,), jnp.int32),
            pltpu.VMEM((ECH,), jnp.int32),
            pltpu.VMEM((ECH,), jnp.int32),
            pltpu.VMEM((ECH,), jnp.int32),
            pltpu.VMEM((NBP,), jnp.int32),
            pltpu.VMEM((NBP,), jnp.int32),
            pltpu.VMEM((NBP,), jnp.int32),
            pltpu.VMEM((NBP,), jnp.int32),
            pltpu.VMEM((16,), jnp.int32),
            pltpu.VMEM((512,), jnp.int32),
            pltpu.VMEM((512,), jnp.float32),
            pltpu.VMEM((512,), jnp.float32),
            pltpu.VMEM((512,), jnp.float32),
            pltpu.VMEM_SHARED((ACCW,), jnp.float32),
            pltpu.SemaphoreType.DMA,
        ],
    )
    def k(src_hbm, dst_hbm, out_hbm, keyA, keyB, srcb, dstb, hidx,
          baseA, wptrA, baseB, wptrB, tmp16, idx512, ones512, zeros512,
          gbuf512, acc, sem):
        cid = lax.axis_index("c")
        sid = lax.axis_index("s")
        wid = cid * NS + sid
        iota = lax.iota(jnp.int32, 16)
        ones16 = jnp.ones((16,), jnp.float32)
        zero16 = jnp.zeros((16,), jnp.float32)

        def fill512(ref, vec):
            def fb(i, _):
                ref[pl.ds(i * 16, 16)] = vec
                return 0
            lax.fori_loop(0, 32, fb, 0)

        fill512(ones512, ones16)
        fill512(zeros512, zero16)

        z0 = sid * ZPT

        def zc(i, _):
            pltpu.sync_copy(zeros512, acc.at[pl.ds(z0 + i * 512, 512)])
            return 0

        lax.fori_loop(0, ZPT // 512, zc, 0)
        pltpu.sync_copy(zeros512.at[pl.ds(0, 128)],
                        acc.at[pl.ds(z0 + (ZPT // 512) * 512, 128)])

        @pl.when(sid == NS - 1)
        def _():
            pltpu.sync_copy(zeros512.at[pl.ds(0, 64)],
                            acc.at[pl.ds(NS * ZPT, 64)])

        plsc.subcore_barrier()

        ebase = sid * EPT2

        def hchunk(ci, _):
            off = ebase + ci * ECH
            pltpu.sync_copy(src_hbm.at[pl.ds(off, ECH)], srcb)
            pltpu.sync_copy(dst_hbm.at[pl.ds(off, ECH)], dstb)

            def hv(j, _):
                s16 = srcb[pl.ds(j * 16, 16)]
                hidx[pl.ds(j * 16, 16)] = (
                    lax.shift_right_logical(s16, SHIFT) + sid * 256)
                return 0

            lax.fori_loop(0, ECH // 16, hv, 0)
            pltpu.sync_copy(ones512, acc.at[hidx.at[pl.ds(0, 512)]], add=True)
            pltpu.sync_copy(ones512,
                            acc.at[hidx.at[pl.ds(512, 512)]], add=True)
            pltpu.sync_copy(ones512,
                            acc.at[hidx.at[pl.ds(1024, 512)]], add=True)
            pltpu.sync_copy(ones512,
                            acc.at[hidx.at[pl.ds(1536, 512)]], add=True)

            def hv2(j, _):
                d16 = dstb[pl.ds(j * 16, 16)]
                hidx[pl.ds(j * 16, 16)] = (
                    lax.shift_right_logical(d16, SHIFT) + sid * 256 + 128)
                return 0

            lax.fori_loop(0, ECH // 16, hv2, 0)
            pltpu.sync_copy(ones512, acc.at[hidx.at[pl.ds(0, 512)]], add=True)
            pltpu.sync_copy(ones512,
                            acc.at[hidx.at[pl.ds(512, 512)]], add=True)
            pltpu.sync_copy(ones512,
                            acc.at[hidx.at[pl.ds(1024, 512)]], add=True)
            pltpu.sync_copy(ones512,
                            acc.at[hidx.at[pl.ds(1536, 512)]], add=True)
            return 0

        lax.fori_loop(0, NECH, hchunk, 0)

        pltpu.sync_copy(acc.at[pl.ds(sid * 256, 128)], gbuf512.at[pl.ds(0, 128)])
        pltpu.sync_copy(acc.at[pl.ds(sid * 256 + 128, 128)],
                        gbuf512.at[pl.ds(128, 128)])

        def mkbase(goff, base_ref, wptr_ref):
            def blk(bi, run):
                h = gbuf512[pl.ds(goff + bi * 16, 16)].astype(jnp.int32)
                hp = lax.shift_left(
                    lax.shift_right_logical(h + 15, 4), 4)
                ic = plsc.cumsum(hp)
                excl = ic - hp + run
                base_ref[pl.ds(bi * 16, 16)] = excl
                wptr_ref[pl.ds(bi * 16, 16)] = excl
                return run + ic[15]
            lax.fori_loop(0, NBP // 16, blk, 0)

        mkbase(0, baseA, wptrA)
        mkbase(128, baseB, wptrB)

        pltpu.sync_copy(zeros512.at[pl.ds(0, 256)],
                        acc.at[pl.ds(sid * 256, 256)])

        def place(b16, k16, wptr_ref):
            sb, sk = plsc.sort_key_val(b16, k16)
            tmp16[...] = sb
            prev = plsc.load_gather(tmp16, [jnp.maximum(iota - 1, 0)])
            nxt = plsc.load_gather(tmp16, [jnp.minimum(iota + 1, 15)])
            is_start = (iota == 0) | (sb != prev)
            is_end = (iota == 15) | (sb != nxt)
            start_pos = plsc.cummax(jnp.where(is_start, iota, 0))
            rank = iota - start_pos
            pos = plsc.load_gather(wptr_ref, [sb]) + rank
            plsc.store_scatter(wptr_ref, [sb], pos + 1, mask=is_end)
            return pos, sk

        def pchunk(ci, _):
            off = ebase + ci * ECH
            pltpu.sync_copy(src_hbm.at[pl.ds(off, ECH)], srcb)
            pltpu.sync_copy(dst_hbm.at[pl.ds(off, ECH)], dstb)

            def pv(j, _):
                s16 = srcb[pl.ds(j * 16, 16)]
                d16 = dstb[pl.ds(j * 16, 16)]
                posA, skA = place(lax.shift_right_logical(s16, SHIFT),
                                  s16 * N + d16, wptrA)
                plsc.store_scatter(keyA, [posA], skA)
                posB, skB = place(lax.shift_right_logical(d16, SHIFT),
                                  d16 * N + s16, wptrB)
                plsc.store_scatter(keyB, [posB], skB)
                return 0

            lax.fori_loop(0, ECH // 16, pv, 0)
            return 0

        lax.fori_loop(0, NECH, pchunk, 0)
        plsc.subcore_barrier()

        def rd(ref, s):
            return plsc.load_gather(
                ref, [jnp.zeros((16,), jnp.int32) + s])[0]

        def scatter_bucket(key_ref, blo, n, sflat, valbuf, add):
            nch = (n + 511) // 512

            def ch(ci, _):
                coff = pl.multiple_of(blo + ci * 512, 16)
                cbase = ci * 512

                def cv(j, _):
                    k16 = key_ref[pl.ds(coff + j * 16, 16)]
                    valid = (cbase + j * 16 + iota) < n
                    idx512[pl.ds(j * 16, 16)] = jnp.where(
                        valid, k16 - sflat, DUMP)
                    return 0

                lax.fori_loop(0, 32, cv, 0)
                pltpu.sync_copy(valbuf, acc.at[idx512], add=add)
                return 0

            lax.fori_loop(0, nch, ch, 0)

        def stripe(i, part):
            s = i * NC + cid
            live = s < NSTRIPE

            @pl.when(live)
            def _():
                blo = rd(baseA, s)
                n = rd(wptrA, s) - blo
                scatter_bucket(keyA, blo, n, s * (R * N), ones512, True)

            plsc.subcore_barrier()

            def gather_sum(part0):
                blo = rd(baseB, s)
                n = rd(wptrB, s) - blo
                nch = (n + 511) // 512

                def ch(ci, p):
                    coff = pl.multiple_of(blo + ci * 512, 16)
                    cbase = ci * 512

                    def cv(j, _):
                        k16 = keyB[pl.ds(coff + j * 16, 16)]
                        valid = (cbase + j * 16 + iota) < n
                        idx512[pl.ds(j * 16, 16)] = jnp.where(
                            valid, k16 - s * (R * N), DUMP)
                        return 0

                    lax.fori_loop(0, 32, cv, 0)
                    pltpu.sync_copy(acc.at[idx512], gbuf512)

                    def av(j, q):
                        g16 = gbuf512[pl.ds(j * 16, 16)]
                        valid = (cbase + j * 16 + iota) < n
                        return q + jnp.where(valid, g16, 0.0)

                    return lax.fori_loop(0, 32, av, p)

                return lax.fori_loop(0, nch, ch, part0)

            part = lax.cond(live, gather_sum, lambda p: p, part)
            plsc.subcore_barrier()

            @pl.when(live)
            def _():
                blo = rd(baseA, s)
                n = rd(wptrA, s) - blo
                scatter_bucket(keyA, blo, n, s * (R * N), zeros512, False)

            plsc.subcore_barrier()
            return part

        part = lax.fori_loop(0, NITER, stripe, jnp.zeros((16,), jnp.float32))

        gbuf512[pl.ds(0, 16)] = part
        pltpu.sync_copy(gbuf512.at[pl.ds(0, 16)],
                        out_hbm.at[pl.ds(wid * 16, 16)])

    return k


_BLK = 1000
_GRID = N // _BLK


def _row_spec(d):
    return pl.BlockSpec((_BLK, d), lambda i: (i, 0))


def _full_spec(r, c):
    return pl.BlockSpec((r, c), lambda i: (0, 0))


def _tc_a_body(x_ref, w_ref, d0_ref, d1_ref, dinv_ref, h_ref, g_ref):
    deg = d0_ref[...] + d1_ref[...] + 1.0
    dinv = lax.rsqrt(deg)
    dinv_ref[...] = dinv
    h = jnp.dot(x_ref[...], w_ref[...], preferred_element_type=jnp.float32)
    h_ref[...] = h
    g_ref[...] = h * dinv


def _tc_a(x, w1, d0, d1):
    return pl.pallas_call(
        _tc_a_body,
        grid=(_GRID,),
        in_specs=[_row_spec(D_IN), _full_spec(D_IN, D_H),
                  _row_spec(1), _row_spec(1)],
        out_specs=[_row_spec(1), _row_spec(D_H), _row_spec(D_H)],
        out_shape=[
            jax.ShapeDtypeStruct((N, 1), jnp.float32),
            jax.ShapeDtypeStruct((N, D_H), jnp.float32),
            jax.ShapeDtypeStruct((N, D_H), jnp.float32),
        ],
    )(x, w1, d0, d1)


def _tc_mid_body(h_ref, p0_ref, p1_ref, dinv_ref, b_ref, w_ref, hn_ref, gn_ref):
    dinv = dinv_ref[...]
    h = h_ref[...]
    z = dinv * (p0_ref[...] + p1_ref[...]) + (dinv * dinv) * h + b_ref[...]
    z = jnp.maximum(z, 0.0)
    hn = jnp.dot(z, w_ref[...], preferred_element_type=jnp.float32)
    hn_ref[...] = hn
    gn_ref[...] = hn * dinv


def _tc_mid(h, p0, p1, dinv, b, w, d_out):
    return pl.pallas_call(
        _tc_mid_body,
        grid=(_GRID,),
        in_specs=[_row_spec(D_H), _row_spec(D_H), _row_spec(D_H),
                  _row_spec(1), _full_spec(1, D_H), _full_spec(D_H, d_out)],
        out_specs=[_row_spec(d_out), _row_spec(d_out)],
        out_shape=[
            jax.ShapeDtypeStruct((N, d_out), jnp.float32),
            jax.ShapeDtypeStruct((N, d_out), jnp.float32),
        ],
    )(h, p0, p1, dinv, b, w)


def _tc_fin_body(h_ref, p0_ref, p1_ref, dinv_ref, b_ref, o_ref):
    dinv = dinv_ref[...]
    h = h_ref[...]
    z = dinv * (p0_ref[...] + p1_ref[...]) + (dinv * dinv) * h + b_ref[...]
    z = z[:, :D_OUT]
    m = jnp.max(z, axis=1, keepdims=True)
    s = jnp.sum(jnp.exp(z - m), axis=1, keepdims=True)
    o_ref[...] = z - m - jnp.log(s)


def _tc_fin(h, p0, p1, dinv, b):
    return pl.pallas_call(
        _tc_fin_body,
        grid=(_GRID,),
        in_specs=[_row_spec(D_H), _row_spec(D_H), _row_spec(D_H),
                  _row_spec(1), _full_spec(1, D_H)],
        out_specs=_row_spec(D_OUT),
        out_shape=jax.ShapeDtypeStruct((N, D_OUT), jnp.float32),
    )(h, p0, p1, dinv, b)


def kernel(x, edge_index, W1, b1, W2, b2, W3, b3, lap_weight):
    del lap_weight
    src = edge_index[0].astype(jnp.int32)
    dst = edge_index[1].astype(jnp.int32)

    degp = _sc_deg()(dst)
    d0 = degp[:N].reshape(N, 1)
    d1 = degp[N:].reshape(N, 1)

    dinv, h1, g1 = _tc_a(x, W1, d0, d1)

    w3p = jnp.pad(W3, ((0, 0), (0, D_H - D_OUT)))
    b3p = jnp.pad(b3, (0, D_H - D_OUT))

    agg_h = _sc_agg(D_H)
    p = agg_h(g1, src, dst)
    h2, g2 = _tc_mid(h1, p[:N], p[N:], dinv, b1.reshape(1, D_H), W2, D_H)
    p = agg_h(g2, src, dst)
    h3, g3 = _tc_mid(h2, p[:N], p[N:], dinv, b2.reshape(1, D_H), w3p, D_H)
    p = agg_h(g3, src, dst)
    out = _tc_fin(h3, p[:N], p[N:], dinv, b3p.reshape(1, D_H))

    keys = src * N + dst
    rev = dst * N + src
    sk = jnp.sort(keys)
    cnt = jnp.searchsorted(sk, rev, side="right") - jnp.searchsorted(
        sk, rev, side="left")
    reg = jnp.sum(cnt).astype(jnp.float32)
    return (out, reg)
